# baseline (speedup 1.0000x reference)
import functools

import jax
import jax.numpy as jnp
from jax import lax
from jax.experimental import pallas as pl
from jax.experimental.pallas import tpu as pltpu
from jax.experimental.pallas import tpu_sc as plsc

N = 10000
E = 160000
EN = E + N
F = 256
HID = 256
C = 64
NEG = 0.2

NC = 2
NS = 16
L = 16

HALF = N // NC
DUMP = HALF
AGG_ROWS = 5120
ROWS_PER_TILE = AGG_ROWS // NS
DEN_PAD = 5120
EN_PAD = 170240
STRIPE = EN_PAD // NS
NCHUNKS = 5
CHUNK = STRIPE // NCHUNKS
B = 16
G = 4
CAP = 2304


def _tc1_body(x_ref, w_ref, wab_ref, h_ref, ab_ref):
    xb = x_ref[...]
    h_ref[...] = jnp.dot(xb, w_ref[...], preferred_element_type=jnp.float32)
    ab_ref[...] = jnp.dot(xb, wab_ref[...], preferred_element_type=jnp.float32)


_tc1 = pl.pallas_call(
    _tc1_body,
    grid=(10,),
    in_specs=[
        pl.BlockSpec((N // 10, F), lambda i: (i, 0)),
        pl.BlockSpec((F, HID), lambda i: (0, 0)),
        pl.BlockSpec((F, 128), lambda i: (0, 0)),
    ],
    out_specs=[
        pl.BlockSpec((N // 10, HID), lambda i: (i, 0)),
        pl.BlockSpec((N // 10, 128), lambda i: (i, 0)),
    ],
    out_shape=[
        jax.ShapeDtypeStruct((N, HID), jnp.float32),
        jax.ShapeDtypeStruct((N, 128), jnp.float32),
    ],
)


_sc_mesh = plsc.VectorSubcoreMesh(core_axis_name="c", subcore_axis_name="s")


@functools.partial(
    pl.kernel,
    mesh=_sc_mesh,
    compiler_params=pltpu.CompilerParams(use_tc_tiling_on_sc=False,
                                         needs_layout_passes=False),
    out_type=[
        jax.ShapeDtypeStruct((NC, AGG_ROWS, HID), jnp.float32),
        jax.ShapeDtypeStruct((NC, NS, DEN_PAD), jnp.float32),
    ],
    scratch_types=[
        pltpu.VMEM((CHUNK,), jnp.int32),
        pltpu.VMEM((CHUNK,), jnp.int32),
        pltpu.VMEM((CAP,), jnp.int32),
        pltpu.VMEM((CAP,), jnp.int32),
        pltpu.VMEM((CAP,), jnp.float32),
        pltpu.VMEM((N,), jnp.float32),
        pltpu.VMEM((DEN_PAD,), jnp.float32),
        pltpu.VMEM((DEN_PAD,), jnp.float32),
        *([pltpu.VMEM((B, HID), jnp.float32)] * G),
        *([pltpu.VMEM((B,), jnp.int32)] * G),
        pltpu.VMEM_SHARED((AGG_ROWS, HID), jnp.float32),
        *([pltpu.SemaphoreType.DMA] * G),
    ],
)
def _sc_gat(h_hbm, asrc_hbm, adst_hbm, src_hbm, dst_hbm,
            agg_out, den_out,
            stage_src, stage_dst, src_l, dst_l, e_l,
            asrc_v, adst_v, denp_v,
            rb0, rb1, rb2, rb3, db0, db1, db2, db3,
            agg_s, sg0, sg1, sg2, sg3):
    bufs = (rb0, rb1, rb2, rb3)
    dbs = (db0, db1, db2, db3)
    sems = (sg0, sg1, sg2, sg3)
    cid = lax.axis_index("c")
    sid = lax.axis_index("s")
    lo = cid * HALF

    pltpu.sync_copy(asrc_hbm, asrc_v)

    zf = jnp.zeros((L,), jnp.float32)
    zi = jnp.zeros((L,), jnp.int32)
    dv = jnp.full((L,), DUMP, jnp.int32)

    def _zadst(i, _):
        adst_v[pl.ds(i * L, L)] = zf
        return 0
    lax.fori_loop((HALF - 8) // L, DEN_PAD // L, _zadst, 0)
    pltpu.sync_copy(adst_hbm.at[pl.ds(lo, HALF)], adst_v.at[pl.ds(0, HALF)])

    def _zden(i, _):
        denp_v[pl.ds(i * L, L)] = zf
        return 0
    lax.fori_loop(0, DEN_PAD // L, _zden, 0)

    def _zrow(i, _):
        rb0[i // (HID // L), pl.ds((i % (HID // L)) * L, L)] = zf
        return 0
    lax.fori_loop(0, B * (HID // L), _zrow, 0)
    base = sid * ROWS_PER_TILE
    for k in range(ROWS_PER_TILE // B):
        pltpu.sync_copy(rb0, agg_s.at[pl.ds(base + k * B, B)])
    plsc.subcore_barrier()

    def _chunk(q, _):
        cbase = sid * STRIPE + q * CHUNK
        with jax.named_scope("sc_stage"):
            pltpu.sync_copy(src_hbm.at[pl.ds(cbase, CHUNK)], stage_src)
            pltpu.sync_copy(dst_hbm.at[pl.ds(cbase, CHUNK)], stage_dst)

        def _init(i, _):
            src_l[pl.ds(i * L, L)] = zi
            dst_l[pl.ds(i * L, L)] = dv
            return 0
        with jax.named_scope("sc_init"):
            lax.fori_loop(0, CAP // L, _init, 0)

        def _comp(i, off):
            s = stage_src[pl.ds(i * L, L)]
            d = stage_dst[pl.ds(i * L, L)]
            m = (d >= lo) & (d < lo + HALF)
            pos = off + plsc.cumsum(m.astype(jnp.int32)) - 1
            plsc.store_scatter(src_l, [pos], s, mask=m)
            plsc.store_scatter(dst_l, [pos], d - lo, mask=m)
            return jnp.max(pos) + 1
        with jax.named_scope("sc_comp"):
            n_e = lax.fori_loop(0, CHUNK // L, _comp, jnp.int32(0))

        def _edge(i, _):
            s = src_l[pl.ds(i * L, L)]
            dl = dst_l[pl.ds(i * L, L)]
            lg = plsc.load_gather(asrc_v, [s]) + plsc.load_gather(adst_v, [dl])
            lg = jnp.where(lg >= 0, lg, lg * NEG)
            ev = jnp.exp(lg)
            e_l[pl.ds(i * L, L)] = ev
            plsc.addupdate_scatter(denp_v, [dl], ev)
            return 0
        with jax.named_scope("sc_edge"):
            lax.fori_loop(0, (n_e + L - 1) // L, _edge, 0)

        ngroups = (n_e + G * B - 1) // (G * B)
        for u in range(G):
            pltpu.async_copy(
                h_hbm.at[src_l.at[pl.ds(u * B, B)]], bufs[u], sems[u])

        def _group(g, _):
            for u in range(G):
                b = g * G + u
                buf, db = bufs[u], dbs[u]
                with jax.named_scope("hv_wait"):
                    pltpu.make_async_copy(
                        h_hbm.at[pl.ds(0, B)], buf, sems[u]).wait()

                with jax.named_scope("hv_scale"):
                    @plsc.parallel_loop(0, B)
                    def _scale(r):
                        av = plsc.load_gather(e_l, [jnp.full((L,), b * B + r,
                                                             jnp.int32)])
                        for c in range(HID // L):
                            buf[r, pl.ds(c * L, L)] = buf[r, pl.ds(c * L, L)] * av
                    for c in range(B // L):
                        db[pl.ds(c * L, L)] = dst_l[pl.ds(b * B + c * L, L)]
                with jax.named_scope("hv_scatter"):
                    pltpu.sync_copy(buf, agg_s.at[db], add=True)

                pltpu.async_copy(
                    h_hbm.at[src_l.at[pl.ds((b + G) * B, B)]],
                    buf, sems[u])
            return 0
        with jax.named_scope("sc_heavy"):
            lax.fori_loop(0, ngroups, _group, 0)
            for u in range(G):
                pltpu.make_async_copy(
                    h_hbm.at[pl.ds(0, B)], bufs[u], sems[u]).wait()
        return 0
    lax.fori_loop(0, NCHUNKS, _chunk, 0)

    pltpu.sync_copy(denp_v, den_out.at[cid, sid])
    plsc.subcore_barrier()

    pltpu.sync_copy(agg_s.at[pl.ds(base, ROWS_PER_TILE)],
                    agg_out.at[cid, pl.ds(base, ROWS_PER_TILE)])


def _tc2_body(agg_ref, den_ref, bg_ref, wo_ref, bo_ref, hr_ref, z_ref):
    dsum = jnp.sum(den_ref[0], axis=0)
    a = agg_ref[0] / (dsum[:, None] + 1e-16) + bg_ref[...][None, :]
    hr = jnp.maximum(a, 0.0)
    hr_ref[0] = hr
    z_ref[0] = jnp.dot(hr, wo_ref[...],
                       preferred_element_type=jnp.float32) + bo_ref[...][None, :]


_BR = 1280

_tc2 = pl.pallas_call(
    _tc2_body,
    grid=(NC, 4),
    in_specs=[
        pl.BlockSpec((1, _BR, HID), lambda h, j: (h, j, 0)),
        pl.BlockSpec((1, NS, _BR), lambda h, j: (h, 0, j)),
        pl.BlockSpec((HID,), lambda h, j: (0,)),
        pl.BlockSpec((HID, C), lambda h, j: (0, 0)),
        pl.BlockSpec((C,), lambda h, j: (0,)),
    ],
    out_specs=[
        pl.BlockSpec((1, _BR, HID), lambda h, j: (h, j, 0)),
        pl.BlockSpec((1, _BR, C), lambda h, j: (h, j, 0)),
    ],
    out_shape=[
        jax.ShapeDtypeStruct((NC, HALF, HID), jnp.float32),
        jax.ShapeDtypeStruct((NC, HALF, C), jnp.float32),
    ],
)


def kernel(x, edge_index, W, att_src, att_dst, bias_gat, W_out, b_out):
    wab = jnp.zeros((F, 128), jnp.float32)
    wab = wab.at[:, 0].set(W @ att_src).at[:, 1].set(W @ att_dst)

    loop = jnp.arange(N, dtype=jnp.int32)
    pad = EN_PAD - EN
    src = jnp.concatenate([edge_index[0], loop,
                           jnp.zeros((pad,), jnp.int32)])
    dst = jnp.concatenate([edge_index[1], loop,
                           jnp.full((pad,), -1, jnp.int32)])

    h, ab = _tc1(x, W, wab)
    agg2, den2 = _sc_gat(h, ab[:, 0], ab[:, 1], src, dst)
    hr2, z2 = _tc2(agg2, den2, bias_gat, W_out, b_out)
    return (hr2.reshape(N, HID), z2.reshape(N, C))


# speedup vs baseline: 1.7783x; 1.7783x over previous
import functools

import jax
import jax.numpy as jnp
from jax import lax
from jax.experimental import pallas as pl
from jax.experimental.pallas import tpu as pltpu
from jax.experimental.pallas import tpu_sc as plsc

N = 10000
E = 160000
EN = E + N
F = 256
HID = 256
C = 64
NEG = 0.2

NC = 2
NS = 16
L = 16

HALF = N // NC
DUMP = HALF
AGG_ROWS = 5120
ROWS_PER_TILE = AGG_ROWS // NS
DEN_PAD = 5120
EN_PAD = 170240
STRIPE = EN_PAD // NS
NCHUNKS = 5
CHUNK = STRIPE // NCHUNKS
B = 32
G = 2
CAP = 2176


def _tc1_body(x_ref, w_ref, wab_ref, h_ref, ab_ref):
    xb = x_ref[...]
    h_ref[...] = jnp.dot(xb, w_ref[...], preferred_element_type=jnp.float32)
    ab_ref[...] = jnp.dot(xb, wab_ref[...], preferred_element_type=jnp.float32)


_tc1 = pl.pallas_call(
    _tc1_body,
    grid=(10,),
    in_specs=[
        pl.BlockSpec((N // 10, F), lambda i: (i, 0)),
        pl.BlockSpec((F, HID), lambda i: (0, 0)),
        pl.BlockSpec((F, 128), lambda i: (0, 0)),
    ],
    out_specs=[
        pl.BlockSpec((N // 10, HID), lambda i: (i, 0)),
        pl.BlockSpec((N // 10, 128), lambda i: (i, 0)),
    ],
    out_shape=[
        jax.ShapeDtypeStruct((N, HID), jnp.float32),
        jax.ShapeDtypeStruct((N, 128), jnp.float32),
    ],
)


_sc_mesh = plsc.VectorSubcoreMesh(core_axis_name="c", subcore_axis_name="s")


@functools.partial(
    pl.kernel,
    mesh=_sc_mesh,
    compiler_params=pltpu.CompilerParams(use_tc_tiling_on_sc=False,
                                         needs_layout_passes=False),
    out_type=[
        jax.ShapeDtypeStruct((NC, AGG_ROWS, HID), jnp.float32),
        jax.ShapeDtypeStruct((NC, NS, DEN_PAD), jnp.float32),
    ],
    scratch_types=[
        pltpu.VMEM((CHUNK,), jnp.int32),
        pltpu.VMEM((CHUNK,), jnp.int32),
        pltpu.VMEM((CAP,), jnp.int32),
        pltpu.VMEM((CAP,), jnp.int32),
        pltpu.VMEM((CAP,), jnp.float32),
        pltpu.VMEM((N,), jnp.float32),
        pltpu.VMEM((DEN_PAD,), jnp.float32),
        pltpu.VMEM((DEN_PAD,), jnp.float32),
        *([pltpu.VMEM((B, HID), jnp.float32)] * G),
        *([pltpu.VMEM((B,), jnp.int32)] * G),
        pltpu.VMEM_SHARED((AGG_ROWS, HID), jnp.float32),
        *([pltpu.SemaphoreType.DMA] * G),
        pltpu.SemaphoreType.DMA,
    ],
)
def _sc_gat(h_hbm, asrc_hbm, adst_hbm, src_hbm, dst_hbm,
            agg_out, den_out,
            stage_src, stage_dst, src_l, dst_l, e_l,
            asrc_v, adst_v, denp_v,
            rb0, rb1, db0, db1,
            agg_s, sg0, sg1, sem_s):
    bufs = (rb0, rb1)
    dbs = (db0, db1)
    sems = (sg0, sg1)
    cid = lax.axis_index("c")
    sid = lax.axis_index("s")
    lo = cid * HALF

    pltpu.sync_copy(asrc_hbm, asrc_v)

    zf = jnp.zeros((L,), jnp.float32)
    zi = jnp.zeros((L,), jnp.int32)
    dv = jnp.full((L,), DUMP, jnp.int32)

    def _zadst(i, _):
        adst_v[pl.ds(i * L, L)] = zf
        return 0
    lax.fori_loop((HALF - 8) // L, DEN_PAD // L, _zadst, 0)
    pltpu.sync_copy(adst_hbm.at[pl.ds(lo, HALF)], adst_v.at[pl.ds(0, HALF)])

    def _zden(i, _):
        denp_v[pl.ds(i * L, L)] = zf
        return 0
    lax.fori_loop(0, DEN_PAD // L, _zden, 0)

    def _zrow(i, _):
        rb0[i // (HID // L), pl.ds((i % (HID // L)) * L, L)] = zf
        return 0
    lax.fori_loop(0, B * (HID // L), _zrow, 0)
    base = sid * ROWS_PER_TILE
    for k in range(ROWS_PER_TILE // B):
        pltpu.sync_copy(rb0, agg_s.at[pl.ds(base + k * B, B)])
    plsc.subcore_barrier()

    def _chunk(q, _):
        cbase = sid * STRIPE + q * CHUNK
        with jax.named_scope("sc_stage"):
            pltpu.sync_copy(src_hbm.at[pl.ds(cbase, CHUNK)], stage_src)
            pltpu.sync_copy(dst_hbm.at[pl.ds(cbase, CHUNK)], stage_dst)

        def _init(i, _):
            src_l[pl.ds(i * L, L)] = zi
            dst_l[pl.ds(i * L, L)] = dv
            return 0
        with jax.named_scope("sc_init"):
            lax.fori_loop(0, CAP // L, _init, 0)

        def _comp(i, off):
            s = stage_src[pl.ds(i * L, L)]
            d = stage_dst[pl.ds(i * L, L)]
            m = (d >= lo) & (d < lo + HALF)
            pos = off + plsc.cumsum(m.astype(jnp.int32)) - 1
            plsc.store_scatter(src_l, [pos], s, mask=m)
            plsc.store_scatter(dst_l, [pos], d - lo, mask=m)
            return jnp.max(pos) + 1
        with jax.named_scope("sc_comp"):
            n_e = lax.fori_loop(0, CHUNK // L, _comp, jnp.int32(0))

        def _edge(i, _):
            s = src_l[pl.ds(i * L, L)]
            dl = dst_l[pl.ds(i * L, L)]
            lg = plsc.load_gather(asrc_v, [s]) + plsc.load_gather(adst_v, [dl])
            lg = jnp.where(lg >= 0, lg, lg * NEG)
            ev = jnp.exp(lg)
            e_l[pl.ds(i * L, L)] = ev
            plsc.addupdate_scatter(denp_v, [dl], ev)
            return 0
        with jax.named_scope("sc_edge"):
            lax.fori_loop(0, (n_e + L - 1) // L, _edge, 0)

        ngroups = (n_e + G * B - 1) // (G * B)

        def _group(g, _):
            cps = []
            for u in range(G):
                b = g * G + u
                cps.append(pltpu.async_copy(
                    h_hbm.at[src_l.at[pl.ds(b * B, B)]], bufs[u], sems[u]))
            for u in range(G):
                b = g * G + u
                buf, db = bufs[u], dbs[u]
                with jax.named_scope("hv_wait"):
                    cps[u].wait()

                with jax.named_scope("hv_scale"):
                    @plsc.parallel_loop(0, B)
                    def _scale(r):
                        av = plsc.load_gather(e_l, [jnp.full((L,), b * B + r,
                                                             jnp.int32)])
                        for c in range(HID // L):
                            buf[r, pl.ds(c * L, L)] = buf[r, pl.ds(c * L, L)] * av
                    for c in range(B // L):
                        db[pl.ds(c * L, L)] = dst_l[pl.ds(b * B + c * L, L)]
                with jax.named_scope("hv_scatter"):
                    pltpu.async_copy(buf, agg_s.at[db], sem_s, add=True)
            for u in range(G):
                with jax.named_scope("hv_drain"):
                    pltpu.make_async_copy(
                        h_hbm.at[pl.ds(0, B)], bufs[u], sem_s).wait()
            return 0
        with jax.named_scope("sc_heavy"):
            lax.fori_loop(0, ngroups, _group, 0)
        return 0
    lax.fori_loop(0, NCHUNKS, _chunk, 0)

    pltpu.sync_copy(denp_v, den_out.at[cid, sid])
    plsc.subcore_barrier()

    pltpu.sync_copy(agg_s.at[pl.ds(base, ROWS_PER_TILE)],
                    agg_out.at[cid, pl.ds(base, ROWS_PER_TILE)])


def _tc2_body(agg_ref, den_ref, bg_ref, wo_ref, bo_ref, hr_ref, z_ref):
    dsum = jnp.sum(den_ref[0], axis=0)
    a = agg_ref[0] / (dsum[:, None] + 1e-16) + bg_ref[...][None, :]
    hr = jnp.maximum(a, 0.0)
    hr_ref[0] = hr
    z_ref[0] = jnp.dot(hr, wo_ref[...],
                       preferred_element_type=jnp.float32) + bo_ref[...][None, :]


_BR = 1280

_tc2 = pl.pallas_call(
    _tc2_body,
    grid=(NC, 4),
    in_specs=[
        pl.BlockSpec((1, _BR, HID), lambda h, j: (h, j, 0)),
        pl.BlockSpec((1, NS, _BR), lambda h, j: (h, 0, j)),
        pl.BlockSpec((HID,), lambda h, j: (0,)),
        pl.BlockSpec((HID, C), lambda h, j: (0, 0)),
        pl.BlockSpec((C,), lambda h, j: (0,)),
    ],
    out_specs=[
        pl.BlockSpec((1, _BR, HID), lambda h, j: (h, j, 0)),
        pl.BlockSpec((1, _BR, C), lambda h, j: (h, j, 0)),
    ],
    out_shape=[
        jax.ShapeDtypeStruct((NC, HALF, HID), jnp.float32),
        jax.ShapeDtypeStruct((NC, HALF, C), jnp.float32),
    ],
)


def kernel(x, edge_index, W, att_src, att_dst, bias_gat, W_out, b_out):
    wab = jnp.zeros((F, 128), jnp.float32)
    wab = wab.at[:, 0].set(W @ att_src).at[:, 1].set(W @ att_dst)

    loop = jnp.arange(N, dtype=jnp.int32)
    pad = EN_PAD - EN
    src = jnp.concatenate([edge_index[0], loop,
                           jnp.zeros((pad,), jnp.int32)])
    dst = jnp.concatenate([edge_index[1], loop,
                           jnp.full((pad,), -1, jnp.int32)])

    h, ab = _tc1(x, W, wab)
    agg2, den2 = _sc_gat(h, ab[:, 0], ab[:, 1], src, dst)
    hr2, z2 = _tc2(agg2, den2, bias_gat, W_out, b_out)
    return (hr2.reshape(N, HID), z2.reshape(N, C))
